# baseline (speedup 1.0000x reference)
import functools

import jax
import jax.numpy as jnp
from jax import lax
from jax.experimental import pallas as pl
from jax.experimental.pallas import tpu as pltpu
from jax.experimental.pallas import tpu_sc as plsc

B = 16384
V = 1000000
D = 64

NC = 2
NS = 16
L = 16
NW = NC * NS
BPW = B // NW
CHUNK = 128
NCH = BPW // CHUNK

_mesh = plsc.VectorSubcoreMesh(core_axis_name="c", subcore_axis_name="s")


@functools.partial(
    pl.kernel,
    out_type=jax.ShapeDtypeStruct((B,), jnp.float32),
    mesh=_mesh,
    compiler_params=pltpu.CompilerParams(
        needs_layout_passes=False, use_tc_tiling_on_sc=False),
    scratch_types=[
        pltpu.VMEM((NCH, CHUNK), jnp.int32),
        pltpu.VMEM((NCH, CHUNK), jnp.int32),
        pltpu.VMEM((BPW, D), jnp.float32),
        pltpu.VMEM((BPW, D), jnp.float32),
        pltpu.VMEM((BPW,), jnp.float32),
        pltpu.SemaphoreType.DMA,
        pltpu.SemaphoreType.DMA,
    ],
)
def _sc_scores(user_ref, item_ref, ut_ref, it_ref, out_ref,
               uidx, iidx, urows, irows, outv, usem, isem):
    wid = lax.axis_index("s") * NC + lax.axis_index("c")

    pltpu.sync_copy(user_ref.at[wid], uidx)
    pltpu.sync_copy(item_ref.at[wid], iidx)
    copies = []
    for j in range(NCH):
        dst = pl.ds(j * CHUNK, CHUNK)
        copies.append(pltpu.async_copy(ut_ref.at[uidx.at[j]], urows.at[dst], usem))
        copies.append(pltpu.async_copy(it_ref.at[iidx.at[j]], irows.at[dst], isem))
    for cp in copies:
        cp.wait()

    lane = lax.iota(jnp.int32, 16)

    def grp_body(g, _):
        base = g * L
        s = jnp.zeros((16,), jnp.float32)
        for k in range(L):
            r = base + k
            acc = urows[r, pl.ds(0, L)] * irows[r, pl.ds(0, L)]
            for cb in range(1, D // L):
                acc = acc + urows[r, pl.ds(cb * L, L)] * irows[r, pl.ds(cb * L, L)]
            s = jnp.where(lane == k, jnp.sum(acc), s)
        outv[pl.ds(base, L)] = 1.0 / (1.0 + jnp.exp(-s))
        return 0

    lax.fori_loop(0, BPW // L, grp_body, 0)
    pltpu.sync_copy(outv, out_ref.at[pl.ds(wid * BPW, BPW)])


def kernel(user, item, user_table, item_table):
    user3 = user.astype(jnp.int32).reshape(NW, NCH, CHUNK)
    item3 = item.astype(jnp.int32).reshape(NW, NCH, CHUNK)
    return _sc_scores(user3, item3, user_table, item_table)
